# Initial kernel scaffold; baseline (speedup 1.0000x reference)
import jax
import jax.numpy as jnp
from jax.experimental import pallas as pl


def kernel(x, edge_index, W1, b1, W2, b2, W3, b3):
    raise NotImplementedError("write your pallas kernel here")


# speedup vs baseline: 5.9685x; 5.9685x over previous
import functools

import jax
import jax.numpy as jnp
from jax import lax
from jax.experimental import pallas as pl
from jax.experimental.pallas import tpu as pltpu
from jax.experimental.pallas import tpu_sc as plsc

N_NODES = 10000
N_PAD = 10240
ROWS_PER_SUB = N_PAD // 16
N_EDGES = 320000
CHUNK = 128
N_CHUNKS = 160
GROUPS = N_CHUNKS // 16
E_PAD = 16 * N_CHUNKS * CHUNK
DUMMY_ROW = N_PAD

F_IN = 256
F_HID = 256
F_OUT = 64


def _sc_mesh():
    return plsc.VectorSubcoreMesh(core_axis_name="c", subcore_axis_name="s")


def _make_deg_kernel():
    @functools.partial(
        pl.kernel,
        out_type=jax.ShapeDtypeStruct((N_PAD,), jnp.float32),
        mesh=_sc_mesh(),
        scratch_types=[
            pltpu.VMEM((N_CHUNKS, CHUNK), jnp.int32),
            pltpu.VMEM((ROWS_PER_SUB,), jnp.float32),
            pltpu.VMEM((CHUNK,), jnp.float32),
            pltpu.VMEM_SHARED((N_PAD + 16,), jnp.float32),
        ],
    )
    def deg_kernel(dst_hbm, deg_out, dst_v, zeros_v, ones_v, acc):
        c = lax.axis_index("c")
        s = lax.axis_index("s")
        base = s * ROWS_PER_SUB

        @pl.when(c == 0)
        def _():
            pltpu.sync_copy(dst_hbm.at[s], dst_v)

            def zinit(i, carry):
                zeros_v[pl.ds(i * 16, 16)] = jnp.zeros((16,), jnp.float32)
                return carry

            lax.fori_loop(0, ROWS_PER_SUB // 16, zinit, 0)

            def oinit(i, carry):
                ones_v[pl.ds(i * 16, 16)] = jnp.ones((16,), jnp.float32)
                return carry

            lax.fori_loop(0, CHUNK // 16, oinit, 0)
            pltpu.sync_copy(zeros_v, acc.at[pl.ds(base, ROWS_PER_SUB)])

        plsc.subcore_barrier()

        @pl.when(c == 0)
        def _():
            def body(j, carry):
                pltpu.sync_copy(ones_v, acc.at[dst_v.at[j]], add=True)
                return carry

            lax.fori_loop(0, N_CHUNKS, body, 0)

        plsc.subcore_barrier()

        @pl.when(c == 0)
        def _():
            pltpu.sync_copy(acc.at[pl.ds(base, ROWS_PER_SUB)],
                            deg_out.at[pl.ds(base, ROWS_PER_SUB)])

    return deg_kernel


def _make_agg_kernel(f_half):
    @functools.partial(
        pl.kernel,
        out_type=[
            jax.ShapeDtypeStruct((N_PAD, f_half), jnp.float32),
            jax.ShapeDtypeStruct((N_PAD, f_half), jnp.float32),
        ],
        mesh=_sc_mesh(),
        scratch_types=[
            pltpu.VMEM((16, CHUNK), jnp.int32),
            pltpu.VMEM((16, CHUNK), jnp.int32),
            pltpu.VMEM((CHUNK, f_half), jnp.float32),
            pltpu.VMEM_SHARED((N_PAD + 8, f_half), jnp.float32),
        ],
    )
    def agg_kernel(h0, h1, src_hbm, dst_hbm, out0, out1,
                   src_v, dst_v, buf, acc):
        c = lax.axis_index("c")
        s = lax.axis_index("s")
        base = s * ROWS_PER_SUB

        @pl.when(c == 0)
        def _():
            pltpu.sync_copy(h0.at[pl.ds(base, ROWS_PER_SUB)],
                            acc.at[pl.ds(base, ROWS_PER_SUB)])

        @pl.when(c == 1)
        def _():
            pltpu.sync_copy(h1.at[pl.ds(base, ROWS_PER_SUB)],
                            acc.at[pl.ds(base, ROWS_PER_SUB)])

        plsc.subcore_barrier()

        def make_loop(h):
            def outer(g, carry):
                pltpu.sync_copy(src_hbm.at[s, pl.ds(g * 16, 16)], src_v)
                pltpu.sync_copy(dst_hbm.at[s, pl.ds(g * 16, 16)], dst_v)

                def inner(j, carry2):
                    pltpu.sync_copy(h.at[src_v.at[j]], buf)
                    pltpu.sync_copy(buf, acc.at[dst_v.at[j]], add=True)
                    return carry2

                lax.fori_loop(0, 16, inner, 0)
                return carry

            lax.fori_loop(0, GROUPS, outer, 0)

        @pl.when(c == 0)
        def _():
            make_loop(h0)

        @pl.when(c == 1)
        def _():
            make_loop(h1)

        plsc.subcore_barrier()

        @pl.when(c == 0)
        def _():
            pltpu.sync_copy(acc.at[pl.ds(base, ROWS_PER_SUB)],
                            out0.at[pl.ds(base, ROWS_PER_SUB)])

        @pl.when(c == 1)
        def _():
            pltpu.sync_copy(acc.at[pl.ds(base, ROWS_PER_SUB)],
                            out1.at[pl.ds(base, ROWS_PER_SUB)])

    return agg_kernel


def _make_agg3_kernel():
    half_chunks = N_CHUNKS // 2
    groups3 = half_chunks // 16

    @functools.partial(
        pl.kernel,
        out_type=[
            jax.ShapeDtypeStruct((N_PAD, 128), jnp.float32),
            jax.ShapeDtypeStruct((N_PAD, 128), jnp.float32),
        ],
        mesh=_sc_mesh(),
        scratch_types=[
            pltpu.VMEM((16, CHUNK), jnp.int32),
            pltpu.VMEM((16, CHUNK), jnp.int32),
            pltpu.VMEM((CHUNK, 128), jnp.float32),
            pltpu.VMEM_SHARED((N_PAD + 8, 128), jnp.float32),
        ],
    )
    def agg3_kernel(h, src_hbm, dst_hbm, out0, out1, src_v, dst_v, buf, acc):
        c = lax.axis_index("c")
        s = lax.axis_index("s")
        base = s * ROWS_PER_SUB

        @pl.when(c == 0)
        def _():
            pltpu.sync_copy(h.at[pl.ds(base, ROWS_PER_SUB)],
                            acc.at[pl.ds(base, ROWS_PER_SUB)])

        @pl.when(c == 1)
        def _():
            def zrow(i, carry):
                for k in range(CHUNK // 16):
                    buf[i, pl.ds(k * 16, 16)] = jnp.zeros((16,), jnp.float32)
                return carry

            lax.fori_loop(0, CHUNK, zrow, 0)
            for r in range(ROWS_PER_SUB // CHUNK):
                pltpu.sync_copy(buf, acc.at[pl.ds(base + r * CHUNK, CHUNK)])

        plsc.subcore_barrier()

        def outer(g, carry):
            off = c * half_chunks + g * 16
            pltpu.sync_copy(src_hbm.at[s, pl.ds(off, 16)], src_v)
            pltpu.sync_copy(dst_hbm.at[s, pl.ds(off, 16)], dst_v)

            def inner(j, carry2):
                pltpu.sync_copy(h.at[src_v.at[j]], buf)
                pltpu.sync_copy(buf, acc.at[dst_v.at[j]], add=True)
                return carry2

            lax.fori_loop(0, 16, inner, 0)
            return carry

        lax.fori_loop(0, groups3, outer, 0)

        plsc.subcore_barrier()

        @pl.when(c == 0)
        def _():
            pltpu.sync_copy(acc.at[pl.ds(base, ROWS_PER_SUB)],
                            out0.at[pl.ds(base, ROWS_PER_SUB)])

        @pl.when(c == 1)
        def _():
            pltpu.sync_copy(acc.at[pl.ds(base, ROWS_PER_SUB)],
                            out1.at[pl.ds(base, ROWS_PER_SUB)])

    return agg3_kernel


_BLK = 1024
_GRID = N_PAD // _BLK


def _dinv(d):
    return jax.lax.rsqrt(d + 1.0)


def _stage_a(x, W1, degr):
    def body(x_ref, w_ref, d_ref, o0, o1):
        dv = _dinv(d_ref[...])
        h = jnp.dot(x_ref[...], w_ref[...], preferred_element_type=jnp.float32)
        o0[...] = h[:, :128] * dv
        o1[...] = h[:, 128:] * dv

    return pl.pallas_call(
        body,
        grid=(_GRID,),
        in_specs=[
            pl.BlockSpec((_BLK, F_IN), lambda i: (i, 0)),
            pl.BlockSpec((F_IN, F_HID), lambda i: (0, 0)),
            pl.BlockSpec((_BLK, 128), lambda i: (i, 0)),
        ],
        out_specs=[
            pl.BlockSpec((_BLK, 128), lambda i: (i, 0)),
            pl.BlockSpec((_BLK, 128), lambda i: (i, 0)),
        ],
        out_shape=[
            jax.ShapeDtypeStruct((N_PAD, 128), jnp.float32),
            jax.ShapeDtypeStruct((N_PAD, 128), jnp.float32),
        ],
    )(x, W1, degr)


def _stage_b(a0, a1, degr, b1, W2):
    def body(a0_ref, a1_ref, d_ref, b_ref, w_ref, oz, o0, o1):
        dv = _dinv(d_ref[...])
        z = jnp.concatenate([a0_ref[...] * dv, a1_ref[...] * dv], axis=1)
        z = jnp.maximum(z + b_ref[...], 0.0)
        oz[...] = z
        g = jnp.dot(z, w_ref[...], preferred_element_type=jnp.float32)
        o0[...] = g[:, :128] * dv
        o1[...] = g[:, 128:] * dv

    return pl.pallas_call(
        body,
        grid=(_GRID,),
        in_specs=[
            pl.BlockSpec((_BLK, 128), lambda i: (i, 0)),
            pl.BlockSpec((_BLK, 128), lambda i: (i, 0)),
            pl.BlockSpec((_BLK, 128), lambda i: (i, 0)),
            pl.BlockSpec((1, F_HID), lambda i: (0, 0)),
            pl.BlockSpec((F_HID, F_HID), lambda i: (0, 0)),
        ],
        out_specs=[
            pl.BlockSpec((_BLK, F_HID), lambda i: (i, 0)),
            pl.BlockSpec((_BLK, 128), lambda i: (i, 0)),
            pl.BlockSpec((_BLK, 128), lambda i: (i, 0)),
        ],
        out_shape=[
            jax.ShapeDtypeStruct((N_PAD, F_HID), jnp.float32),
            jax.ShapeDtypeStruct((N_PAD, 128), jnp.float32),
            jax.ShapeDtypeStruct((N_PAD, 128), jnp.float32),
        ],
    )(a0, a1, degr, b1, W2)


def _stage_c(a0, a1, degr, b2, z, W3):
    def body(a0_ref, a1_ref, d_ref, b_ref, z_ref, w_ref, o):
        dv = _dinv(d_ref[...])
        m = jnp.concatenate([a0_ref[...] * dv, a1_ref[...] * dv], axis=1)
        h2 = jnp.maximum(m + b_ref[...] + z_ref[...], 0.0)
        g = jnp.dot(h2, w_ref[...], preferred_element_type=jnp.float32)
        gs = g * dv[:, :F_OUT]
        o[...] = jnp.concatenate([gs, jnp.zeros_like(gs)], axis=1)

    return pl.pallas_call(
        body,
        grid=(_GRID,),
        in_specs=[
            pl.BlockSpec((_BLK, 128), lambda i: (i, 0)),
            pl.BlockSpec((_BLK, 128), lambda i: (i, 0)),
            pl.BlockSpec((_BLK, 128), lambda i: (i, 0)),
            pl.BlockSpec((1, F_HID), lambda i: (0, 0)),
            pl.BlockSpec((_BLK, F_HID), lambda i: (i, 0)),
            pl.BlockSpec((F_HID, F_OUT), lambda i: (0, 0)),
        ],
        out_specs=pl.BlockSpec((_BLK, 128), lambda i: (i, 0)),
        out_shape=jax.ShapeDtypeStruct((N_PAD, 128), jnp.float32),
    )(a0, a1, degr, b2, z, W3)


def _stage_d(a0, a1, degr, b3):
    def body(a0_ref, a1_ref, d_ref, b_ref, o):
        dv = _dinv(d_ref[...])
        agg = a0_ref[...] + a1_ref[...]
        o[...] = agg[:, :F_OUT] * dv[:, :F_OUT] + b_ref[...]

    return pl.pallas_call(
        body,
        grid=(_GRID,),
        in_specs=[
            pl.BlockSpec((_BLK, 128), lambda i: (i, 0)),
            pl.BlockSpec((_BLK, 128), lambda i: (i, 0)),
            pl.BlockSpec((_BLK, 128), lambda i: (i, 0)),
            pl.BlockSpec((1, F_OUT), lambda i: (0, 0)),
        ],
        out_specs=pl.BlockSpec((_BLK, F_OUT), lambda i: (i, 0)),
        out_shape=jax.ShapeDtypeStruct((N_PAD, F_OUT), jnp.float32),
    )(a0, a1, degr, b3)


_deg_kernel = _make_deg_kernel()
_agg_hid = _make_agg_kernel(F_HID // 2)
_agg3 = _make_agg3_kernel()


def kernel(x, edge_index, W1, b1, W2, b2, W3, b3):
    ei = edge_index.astype(jnp.int32)
    src = ei[0]
    dst = ei[1]
    pad = E_PAD - N_EDGES
    srcp = jnp.concatenate([src, jnp.zeros((pad,), jnp.int32)])
    dstp = jnp.concatenate([dst, jnp.full((pad,), DUMMY_ROW, jnp.int32)])
    src16 = srcp.reshape(16, N_CHUNKS, CHUNK)
    dst16 = dstp.reshape(16, N_CHUNKS, CHUNK)

    xp = jnp.pad(x, ((0, N_PAD - N_NODES), (0, 0)))

    deg = _deg_kernel(dst16)
    degr = jnp.broadcast_to(deg[:, None], (N_PAD, 128))

    h0, h1 = _stage_a(xp, W1, degr)
    a0, a1 = _agg_hid(h0, h1, src16, dst16)
    z, g0, g1 = _stage_b(a0, a1, degr, b1.reshape(1, F_HID), W2)
    a0, a1 = _agg_hid(g0, g1, src16, dst16)
    g3 = _stage_c(a0, a1, degr, b2.reshape(1, F_HID), z, W3)
    a0, a1 = _agg3(g3, src16, dst16)
    out = _stage_d(a0, a1, degr, b3.reshape(1, F_OUT))
    return out[:N_NODES]


# speedup vs baseline: 6.6452x; 1.1134x over previous
import functools

import jax
import jax.numpy as jnp
from jax import lax
from jax.experimental import pallas as pl
from jax.experimental.pallas import tpu as pltpu
from jax.experimental.pallas import tpu_sc as plsc

N_NODES = 10000
N_PAD = 10240
ROWS_PER_SUB = N_PAD // 16
N_EDGES = 320000
CHUNK = 128
N_CHUNKS = 160
GROUPS = N_CHUNKS // 16
E_PAD = 16 * N_CHUNKS * CHUNK
DUMMY_ROW = N_PAD

F_IN = 256
F_HID = 256
F_OUT = 64


def _sc_mesh():
    return plsc.VectorSubcoreMesh(core_axis_name="c", subcore_axis_name="s")


def _make_deg_kernel():
    @functools.partial(
        pl.kernel,
        out_type=jax.ShapeDtypeStruct((N_PAD,), jnp.float32),
        mesh=_sc_mesh(),
        scratch_types=[
            pltpu.VMEM((N_CHUNKS, CHUNK), jnp.int32),
            pltpu.VMEM((ROWS_PER_SUB,), jnp.float32),
            pltpu.VMEM((CHUNK,), jnp.float32),
            pltpu.VMEM_SHARED((N_PAD + 16,), jnp.float32),
        ],
    )
    def deg_kernel(dst_hbm, deg_out, dst_v, zeros_v, ones_v, acc):
        c = lax.axis_index("c")
        s = lax.axis_index("s")
        base = s * ROWS_PER_SUB

        @pl.when(c == 0)
        def _():
            pltpu.sync_copy(dst_hbm.at[s], dst_v)

            def zinit(i, carry):
                zeros_v[pl.ds(i * 16, 16)] = jnp.zeros((16,), jnp.float32)
                return carry

            lax.fori_loop(0, ROWS_PER_SUB // 16, zinit, 0)

            def oinit(i, carry):
                ones_v[pl.ds(i * 16, 16)] = jnp.ones((16,), jnp.float32)
                return carry

            lax.fori_loop(0, CHUNK // 16, oinit, 0)
            pltpu.sync_copy(zeros_v, acc.at[pl.ds(base, ROWS_PER_SUB)])

        plsc.subcore_barrier()

        @pl.when(c == 0)
        def _():
            def body(j, carry):
                pltpu.sync_copy(ones_v, acc.at[dst_v.at[j]], add=True)
                return carry

            lax.fori_loop(0, N_CHUNKS, body, 0)

        plsc.subcore_barrier()

        @pl.when(c == 0)
        def _():
            pltpu.sync_copy(acc.at[pl.ds(base, ROWS_PER_SUB)],
                            deg_out.at[pl.ds(base, ROWS_PER_SUB)])

    return deg_kernel


def _make_agg_kernel(f_half):
    @functools.partial(
        pl.kernel,
        out_type=[
            jax.ShapeDtypeStruct((N_PAD, f_half), jnp.float32),
            jax.ShapeDtypeStruct((N_PAD, f_half), jnp.float32),
        ],
        mesh=_sc_mesh(),
        scratch_types=[
            pltpu.VMEM((16, CHUNK), jnp.int32),
            pltpu.VMEM((16, CHUNK), jnp.int32),
            pltpu.VMEM((2, CHUNK, f_half), jnp.float32),
            pltpu.VMEM_SHARED((N_PAD + 8, f_half), jnp.float32),
            pltpu.SemaphoreType.DMA,
        ],
    )
    def agg_kernel(h0, h1, src_hbm, dst_hbm, out0, out1,
                   src_v, dst_v, buf, acc, ssem):
        c = lax.axis_index("c")
        s = lax.axis_index("s")
        base = s * ROWS_PER_SUB

        @pl.when(c == 0)
        def _():
            pltpu.sync_copy(h0.at[pl.ds(base, ROWS_PER_SUB)],
                            acc.at[pl.ds(base, ROWS_PER_SUB)])

        @pl.when(c == 1)
        def _():
            pltpu.sync_copy(h1.at[pl.ds(base, ROWS_PER_SUB)],
                            acc.at[pl.ds(base, ROWS_PER_SUB)])

        plsc.subcore_barrier()

        def make_loop(h):
            def outer(g, carry):
                pltpu.sync_copy(src_hbm.at[s, pl.ds(g * 16, 16)], src_v)
                pltpu.sync_copy(dst_hbm.at[s, pl.ds(g * 16, 16)], dst_v)

                def inner(k, carry2):
                    kb = k % 2

                    @pl.when(jnp.logical_or(g > 0, k >= 2))
                    def _():
                        pltpu.make_async_copy(
                            buf.at[kb], acc.at[pl.ds(0, CHUNK)], ssem).wait()

                    pltpu.sync_copy(h.at[src_v.at[k]], buf.at[kb])
                    pltpu.async_copy(buf.at[kb], acc.at[dst_v.at[k]], ssem,
                                     add=True)
                    return carry2

                lax.fori_loop(0, 16, inner, 0)
                return carry

            lax.fori_loop(0, GROUPS, outer, 0)
            pltpu.make_async_copy(buf.at[0], acc.at[pl.ds(0, CHUNK)], ssem).wait()
            pltpu.make_async_copy(buf.at[1], acc.at[pl.ds(0, CHUNK)], ssem).wait()

        @pl.when(c == 0)
        def _():
            make_loop(h0)

        @pl.when(c == 1)
        def _():
            make_loop(h1)

        plsc.subcore_barrier()

        @pl.when(c == 0)
        def _():
            pltpu.sync_copy(acc.at[pl.ds(base, ROWS_PER_SUB)],
                            out0.at[pl.ds(base, ROWS_PER_SUB)])

        @pl.when(c == 1)
        def _():
            pltpu.sync_copy(acc.at[pl.ds(base, ROWS_PER_SUB)],
                            out1.at[pl.ds(base, ROWS_PER_SUB)])

    return agg_kernel


def _make_agg3_kernel():
    half_chunks = N_CHUNKS // 2
    groups3 = half_chunks // 16

    @functools.partial(
        pl.kernel,
        out_type=[
            jax.ShapeDtypeStruct((N_PAD, 128), jnp.float32),
            jax.ShapeDtypeStruct((N_PAD, 128), jnp.float32),
        ],
        mesh=_sc_mesh(),
        scratch_types=[
            pltpu.VMEM((16, CHUNK), jnp.int32),
            pltpu.VMEM((16, CHUNK), jnp.int32),
            pltpu.VMEM((2, CHUNK, 128), jnp.float32),
            pltpu.VMEM_SHARED((N_PAD + 8, 128), jnp.float32),
            pltpu.SemaphoreType.DMA,
        ],
    )
    def agg3_kernel(h, src_hbm, dst_hbm, out0, out1, src_v, dst_v, buf, acc,
                    ssem):
        c = lax.axis_index("c")
        s = lax.axis_index("s")
        base = s * ROWS_PER_SUB

        @pl.when(c == 0)
        def _():
            pltpu.sync_copy(h.at[pl.ds(base, ROWS_PER_SUB)],
                            acc.at[pl.ds(base, ROWS_PER_SUB)])

        @pl.when(c == 1)
        def _():
            def zrow(i, carry):
                for k in range(CHUNK // 16):
                    buf[0, i, pl.ds(k * 16, 16)] = jnp.zeros((16,), jnp.float32)
                return carry

            lax.fori_loop(0, CHUNK, zrow, 0)
            for r in range(ROWS_PER_SUB // CHUNK):
                pltpu.sync_copy(buf.at[0], acc.at[pl.ds(base + r * CHUNK, CHUNK)])

        plsc.subcore_barrier()

        def outer(g, carry):
            off = c * half_chunks + g * 16
            pltpu.sync_copy(src_hbm.at[s, pl.ds(off, 16)], src_v)
            pltpu.sync_copy(dst_hbm.at[s, pl.ds(off, 16)], dst_v)

            def inner(k, carry2):
                kb = k % 2

                @pl.when(jnp.logical_or(g > 0, k >= 2))
                def _():
                    pltpu.make_async_copy(
                        buf.at[kb], acc.at[pl.ds(0, CHUNK)], ssem).wait()

                pltpu.sync_copy(h.at[src_v.at[k]], buf.at[kb])
                pltpu.async_copy(buf.at[kb], acc.at[dst_v.at[k]], ssem,
                                 add=True)
                return carry2

            lax.fori_loop(0, 16, inner, 0)
            return carry

        lax.fori_loop(0, groups3, outer, 0)
        pltpu.make_async_copy(buf.at[0], acc.at[pl.ds(0, CHUNK)], ssem).wait()
        pltpu.make_async_copy(buf.at[1], acc.at[pl.ds(0, CHUNK)], ssem).wait()

        plsc.subcore_barrier()

        @pl.when(c == 0)
        def _():
            pltpu.sync_copy(acc.at[pl.ds(base, ROWS_PER_SUB)],
                            out0.at[pl.ds(base, ROWS_PER_SUB)])

        @pl.when(c == 1)
        def _():
            pltpu.sync_copy(acc.at[pl.ds(base, ROWS_PER_SUB)],
                            out1.at[pl.ds(base, ROWS_PER_SUB)])

    return agg3_kernel


_BLK = 1024
_GRID = N_PAD // _BLK


def _dinv(d):
    return jax.lax.rsqrt(d + 1.0)


def _stage_a(x, W1, degr):
    def body(x_ref, w_ref, d_ref, o0, o1):
        dv = _dinv(d_ref[...])
        h = jnp.dot(x_ref[...], w_ref[...], preferred_element_type=jnp.float32)
        o0[...] = h[:, :128] * dv
        o1[...] = h[:, 128:] * dv

    return pl.pallas_call(
        body,
        grid=(_GRID,),
        in_specs=[
            pl.BlockSpec((_BLK, F_IN), lambda i: (i, 0)),
            pl.BlockSpec((F_IN, F_HID), lambda i: (0, 0)),
            pl.BlockSpec((_BLK, 128), lambda i: (i, 0)),
        ],
        out_specs=[
            pl.BlockSpec((_BLK, 128), lambda i: (i, 0)),
            pl.BlockSpec((_BLK, 128), lambda i: (i, 0)),
        ],
        out_shape=[
            jax.ShapeDtypeStruct((N_PAD, 128), jnp.float32),
            jax.ShapeDtypeStruct((N_PAD, 128), jnp.float32),
        ],
    )(x, W1, degr)


def _stage_b(a0, a1, degr, b1, W2):
    def body(a0_ref, a1_ref, d_ref, b_ref, w_ref, oz, o0, o1):
        dv = _dinv(d_ref[...])
        z = jnp.concatenate([a0_ref[...] * dv, a1_ref[...] * dv], axis=1)
        z = jnp.maximum(z + b_ref[...], 0.0)
        oz[...] = z
        g = jnp.dot(z, w_ref[...], preferred_element_type=jnp.float32)
        o0[...] = g[:, :128] * dv
        o1[...] = g[:, 128:] * dv

    return pl.pallas_call(
        body,
        grid=(_GRID,),
        in_specs=[
            pl.BlockSpec((_BLK, 128), lambda i: (i, 0)),
            pl.BlockSpec((_BLK, 128), lambda i: (i, 0)),
            pl.BlockSpec((_BLK, 128), lambda i: (i, 0)),
            pl.BlockSpec((1, F_HID), lambda i: (0, 0)),
            pl.BlockSpec((F_HID, F_HID), lambda i: (0, 0)),
        ],
        out_specs=[
            pl.BlockSpec((_BLK, F_HID), lambda i: (i, 0)),
            pl.BlockSpec((_BLK, 128), lambda i: (i, 0)),
            pl.BlockSpec((_BLK, 128), lambda i: (i, 0)),
        ],
        out_shape=[
            jax.ShapeDtypeStruct((N_PAD, F_HID), jnp.float32),
            jax.ShapeDtypeStruct((N_PAD, 128), jnp.float32),
            jax.ShapeDtypeStruct((N_PAD, 128), jnp.float32),
        ],
    )(a0, a1, degr, b1, W2)


def _stage_c(a0, a1, degr, b2, z, W3):
    def body(a0_ref, a1_ref, d_ref, b_ref, z_ref, w_ref, o):
        dv = _dinv(d_ref[...])
        m = jnp.concatenate([a0_ref[...] * dv, a1_ref[...] * dv], axis=1)
        h2 = jnp.maximum(m + b_ref[...] + z_ref[...], 0.0)
        g = jnp.dot(h2, w_ref[...], preferred_element_type=jnp.float32)
        gs = g * dv[:, :F_OUT]
        o[...] = jnp.concatenate([gs, jnp.zeros_like(gs)], axis=1)

    return pl.pallas_call(
        body,
        grid=(_GRID,),
        in_specs=[
            pl.BlockSpec((_BLK, 128), lambda i: (i, 0)),
            pl.BlockSpec((_BLK, 128), lambda i: (i, 0)),
            pl.BlockSpec((_BLK, 128), lambda i: (i, 0)),
            pl.BlockSpec((1, F_HID), lambda i: (0, 0)),
            pl.BlockSpec((_BLK, F_HID), lambda i: (i, 0)),
            pl.BlockSpec((F_HID, F_OUT), lambda i: (0, 0)),
        ],
        out_specs=pl.BlockSpec((_BLK, 128), lambda i: (i, 0)),
        out_shape=jax.ShapeDtypeStruct((N_PAD, 128), jnp.float32),
    )(a0, a1, degr, b2, z, W3)


def _stage_d(a0, a1, degr, b3):
    def body(a0_ref, a1_ref, d_ref, b_ref, o):
        dv = _dinv(d_ref[...])
        agg = a0_ref[...] + a1_ref[...]
        o[...] = agg[:, :F_OUT] * dv[:, :F_OUT] + b_ref[...]

    return pl.pallas_call(
        body,
        grid=(_GRID,),
        in_specs=[
            pl.BlockSpec((_BLK, 128), lambda i: (i, 0)),
            pl.BlockSpec((_BLK, 128), lambda i: (i, 0)),
            pl.BlockSpec((_BLK, 128), lambda i: (i, 0)),
            pl.BlockSpec((1, F_OUT), lambda i: (0, 0)),
        ],
        out_specs=pl.BlockSpec((_BLK, F_OUT), lambda i: (i, 0)),
        out_shape=jax.ShapeDtypeStruct((N_PAD, F_OUT), jnp.float32),
    )(a0, a1, degr, b3)


_deg_kernel = _make_deg_kernel()
_agg_hid = _make_agg_kernel(F_HID // 2)
_agg3 = _make_agg3_kernel()


def kernel(x, edge_index, W1, b1, W2, b2, W3, b3):
    ei = edge_index.astype(jnp.int32)
    src = ei[0]
    dst = ei[1]
    pad = E_PAD - N_EDGES
    srcp = jnp.concatenate([src, jnp.zeros((pad,), jnp.int32)])
    dstp = jnp.concatenate([dst, jnp.full((pad,), DUMMY_ROW, jnp.int32)])
    src16 = srcp.reshape(16, N_CHUNKS, CHUNK)
    dst16 = dstp.reshape(16, N_CHUNKS, CHUNK)

    xp = jnp.pad(x, ((0, N_PAD - N_NODES), (0, 0)))

    deg = _deg_kernel(dst16)
    degr = jnp.broadcast_to(deg[:, None], (N_PAD, 128))

    h0, h1 = _stage_a(xp, W1, degr)
    a0, a1 = _agg_hid(h0, h1, src16, dst16)
    z, g0, g1 = _stage_b(a0, a1, degr, b1.reshape(1, F_HID), W2)
    a0, a1 = _agg_hid(g0, g1, src16, dst16)
    g3 = _stage_c(a0, a1, degr, b2.reshape(1, F_HID), z, W3)
    a0, a1 = _agg3(g3, src16, dst16)
    out = _stage_d(a0, a1, degr, b3.reshape(1, F_OUT))
    return out[:N_NODES]


# speedup vs baseline: 7.0204x; 1.0565x over previous
import functools

import jax
import jax.numpy as jnp
from jax import lax
from jax.experimental import pallas as pl
from jax.experimental.pallas import tpu as pltpu
from jax.experimental.pallas import tpu_sc as plsc

N_NODES = 10000
N_PAD = 10240
ROWS_PER_SUB = N_PAD // 16
N_EDGES = 320000
CHUNK = 128
N_CHUNKS = 160
GROUPS = N_CHUNKS // 16
E_PAD = 16 * N_CHUNKS * CHUNK
DUMMY_ROW = N_PAD

F_IN = 256
F_HID = 256
F_OUT = 64


def _sc_mesh():
    return plsc.VectorSubcoreMesh(core_axis_name="c", subcore_axis_name="s")


def _edge_pipeline(h, acc, src_hbm, dst_hbm, s, chunk0, ngroups,
                   src_v, dst_v, buf, gsem, ssem):
    def stage(g):
        gb = g % 2
        pltpu.sync_copy(src_hbm.at[s, pl.ds(chunk0 + g * 16, 16)],
                        src_v.at[gb])
        pltpu.sync_copy(dst_hbm.at[s, pl.ds(chunk0 + g * 16, 16)],
                        dst_v.at[gb])

    def wait_scatter(b):
        pltpu.make_async_copy(buf.at[b], acc.at[pl.ds(0, CHUNK)], ssem).wait()

    def wait_gather(b):
        pltpu.make_async_copy(h.at[pl.ds(0, CHUNK)], buf.at[b], gsem).wait()

    stage(0)
    pltpu.async_copy(h.at[src_v.at[0, 0]], buf.at[0], gsem)

    def outer(g, carry):
        gb = g % 2

        def inner(k, carry2):
            kb = k % 2

            @pl.when(k < 15)
            def _():
                @pl.when(g * 16 + k >= 1)
                def _():
                    wait_scatter(1 - kb)

                pltpu.async_copy(h.at[src_v.at[gb, k + 1]], buf.at[1 - kb],
                                 gsem)

            wait_gather(kb)
            pltpu.async_copy(buf.at[kb], acc.at[dst_v.at[gb, k]], ssem,
                             add=True)
            return carry2

        lax.fori_loop(0, 16, inner, 0)

        @pl.when(g + 1 < ngroups)
        def _():
            stage(g + 1)
            wait_scatter(0)
            pltpu.async_copy(h.at[src_v.at[(g + 1) % 2, 0]], buf.at[0], gsem)

        return carry

    lax.fori_loop(0, ngroups, outer, 0)
    wait_scatter(0)
    wait_scatter(1)


def _make_deg_kernel():
    @functools.partial(
        pl.kernel,
        out_type=jax.ShapeDtypeStruct((N_PAD,), jnp.float32),
        mesh=_sc_mesh(),
        scratch_types=[
            pltpu.VMEM((N_CHUNKS, CHUNK), jnp.int32),
            pltpu.VMEM((ROWS_PER_SUB,), jnp.float32),
            pltpu.VMEM((CHUNK,), jnp.float32),
            pltpu.VMEM_SHARED((N_PAD + 16,), jnp.float32),
        ],
    )
    def deg_kernel(dst_hbm, deg_out, dst_v, zeros_v, ones_v, acc):
        c = lax.axis_index("c")
        s = lax.axis_index("s")
        base = s * ROWS_PER_SUB

        @pl.when(c == 0)
        def _():
            pltpu.sync_copy(dst_hbm.at[s], dst_v)

            def zinit(i, carry):
                zeros_v[pl.ds(i * 16, 16)] = jnp.zeros((16,), jnp.float32)
                return carry

            lax.fori_loop(0, ROWS_PER_SUB // 16, zinit, 0)

            def oinit(i, carry):
                ones_v[pl.ds(i * 16, 16)] = jnp.ones((16,), jnp.float32)
                return carry

            lax.fori_loop(0, CHUNK // 16, oinit, 0)
            pltpu.sync_copy(zeros_v, acc.at[pl.ds(base, ROWS_PER_SUB)])

        plsc.subcore_barrier()

        @pl.when(c == 0)
        def _():
            def body(j, carry):
                pltpu.sync_copy(ones_v, acc.at[dst_v.at[j]], add=True)
                return carry

            lax.fori_loop(0, N_CHUNKS, body, 0)

        plsc.subcore_barrier()

        @pl.when(c == 0)
        def _():
            pltpu.sync_copy(acc.at[pl.ds(base, ROWS_PER_SUB)],
                            deg_out.at[pl.ds(base, ROWS_PER_SUB)])

    return deg_kernel


def _make_agg_kernel(f_half):
    @functools.partial(
        pl.kernel,
        out_type=[
            jax.ShapeDtypeStruct((N_PAD, f_half), jnp.float32),
            jax.ShapeDtypeStruct((N_PAD, f_half), jnp.float32),
        ],
        mesh=_sc_mesh(),
        scratch_types=[
            pltpu.VMEM((2, 16, CHUNK), jnp.int32),
            pltpu.VMEM((2, 16, CHUNK), jnp.int32),
            pltpu.VMEM((2, CHUNK, f_half), jnp.float32),
            pltpu.VMEM_SHARED((N_PAD + 8, f_half), jnp.float32),
            pltpu.SemaphoreType.DMA,
            pltpu.SemaphoreType.DMA,
        ],
    )
    def agg_kernel(h0, h1, src_hbm, dst_hbm, out0, out1,
                   src_v, dst_v, buf, acc, gsem, ssem):
        c = lax.axis_index("c")
        s = lax.axis_index("s")
        base = s * ROWS_PER_SUB

        @pl.when(c == 0)
        def _():
            pltpu.sync_copy(h0.at[pl.ds(base, ROWS_PER_SUB)],
                            acc.at[pl.ds(base, ROWS_PER_SUB)])

        @pl.when(c == 1)
        def _():
            pltpu.sync_copy(h1.at[pl.ds(base, ROWS_PER_SUB)],
                            acc.at[pl.ds(base, ROWS_PER_SUB)])

        plsc.subcore_barrier()

        @pl.when(c == 0)
        def _():
            _edge_pipeline(h0, acc, src_hbm, dst_hbm, s, 0, GROUPS,
                           src_v, dst_v, buf, gsem, ssem)

        @pl.when(c == 1)
        def _():
            _edge_pipeline(h1, acc, src_hbm, dst_hbm, s, 0, GROUPS,
                           src_v, dst_v, buf, gsem, ssem)

        plsc.subcore_barrier()

        @pl.when(c == 0)
        def _():
            pltpu.sync_copy(acc.at[pl.ds(base, ROWS_PER_SUB)],
                            out0.at[pl.ds(base, ROWS_PER_SUB)])

        @pl.when(c == 1)
        def _():
            pltpu.sync_copy(acc.at[pl.ds(base, ROWS_PER_SUB)],
                            out1.at[pl.ds(base, ROWS_PER_SUB)])

    return agg_kernel


def _make_agg3_kernel():
    half_chunks = N_CHUNKS // 2
    groups3 = half_chunks // 16

    @functools.partial(
        pl.kernel,
        out_type=[
            jax.ShapeDtypeStruct((N_PAD, 128), jnp.float32),
            jax.ShapeDtypeStruct((N_PAD, 128), jnp.float32),
        ],
        mesh=_sc_mesh(),
        scratch_types=[
            pltpu.VMEM((2, 16, CHUNK), jnp.int32),
            pltpu.VMEM((2, 16, CHUNK), jnp.int32),
            pltpu.VMEM((2, CHUNK, 128), jnp.float32),
            pltpu.VMEM_SHARED((N_PAD + 8, 128), jnp.float32),
            pltpu.SemaphoreType.DMA,
            pltpu.SemaphoreType.DMA,
        ],
    )
    def agg3_kernel(h, src_hbm, dst_hbm, out0, out1, src_v, dst_v, buf, acc,
                    gsem, ssem):
        c = lax.axis_index("c")
        s = lax.axis_index("s")
        base = s * ROWS_PER_SUB

        @pl.when(c == 0)
        def _():
            pltpu.sync_copy(h.at[pl.ds(base, ROWS_PER_SUB)],
                            acc.at[pl.ds(base, ROWS_PER_SUB)])

        @pl.when(c == 1)
        def _():
            def zrow(i, carry):
                for k in range(CHUNK // 16):
                    buf[0, i, pl.ds(k * 16, 16)] = jnp.zeros((16,), jnp.float32)
                return carry

            lax.fori_loop(0, CHUNK, zrow, 0)
            for r in range(ROWS_PER_SUB // CHUNK):
                pltpu.sync_copy(buf.at[0],
                                acc.at[pl.ds(base + r * CHUNK, CHUNK)])

        plsc.subcore_barrier()

        _edge_pipeline(h, acc, src_hbm, dst_hbm, s, c * half_chunks, groups3,
                       src_v, dst_v, buf, gsem, ssem)

        plsc.subcore_barrier()

        @pl.when(c == 0)
        def _():
            pltpu.sync_copy(acc.at[pl.ds(base, ROWS_PER_SUB)],
                            out0.at[pl.ds(base, ROWS_PER_SUB)])

        @pl.when(c == 1)
        def _():
            pltpu.sync_copy(acc.at[pl.ds(base, ROWS_PER_SUB)],
                            out1.at[pl.ds(base, ROWS_PER_SUB)])

    return agg3_kernel


_BLK = 1024
_GRID = N_PAD // _BLK


def _dinv(d):
    return jax.lax.rsqrt(d + 1.0)


def _stage_a(x, W1, degr):
    def body(x_ref, w_ref, d_ref, o0, o1):
        dv = _dinv(d_ref[...])
        h = jnp.dot(x_ref[...], w_ref[...], preferred_element_type=jnp.float32)
        o0[...] = h[:, :128] * dv
        o1[...] = h[:, 128:] * dv

    return pl.pallas_call(
        body,
        grid=(_GRID,),
        in_specs=[
            pl.BlockSpec((_BLK, F_IN), lambda i: (i, 0)),
            pl.BlockSpec((F_IN, F_HID), lambda i: (0, 0)),
            pl.BlockSpec((_BLK, 128), lambda i: (i, 0)),
        ],
        out_specs=[
            pl.BlockSpec((_BLK, 128), lambda i: (i, 0)),
            pl.BlockSpec((_BLK, 128), lambda i: (i, 0)),
        ],
        out_shape=[
            jax.ShapeDtypeStruct((N_PAD, 128), jnp.float32),
            jax.ShapeDtypeStruct((N_PAD, 128), jnp.float32),
        ],
    )(x, W1, degr)


def _stage_b(a0, a1, degr, b1, W2):
    def body(a0_ref, a1_ref, d_ref, b_ref, w_ref, oz, o0, o1):
        dv = _dinv(d_ref[...])
        z = jnp.concatenate([a0_ref[...] * dv, a1_ref[...] * dv], axis=1)
        z = jnp.maximum(z + b_ref[...], 0.0)
        oz[...] = z
        g = jnp.dot(z, w_ref[...], preferred_element_type=jnp.float32)
        o0[...] = g[:, :128] * dv
        o1[...] = g[:, 128:] * dv

    return pl.pallas_call(
        body,
        grid=(_GRID,),
        in_specs=[
            pl.BlockSpec((_BLK, 128), lambda i: (i, 0)),
            pl.BlockSpec((_BLK, 128), lambda i: (i, 0)),
            pl.BlockSpec((_BLK, 128), lambda i: (i, 0)),
            pl.BlockSpec((1, F_HID), lambda i: (0, 0)),
            pl.BlockSpec((F_HID, F_HID), lambda i: (0, 0)),
        ],
        out_specs=[
            pl.BlockSpec((_BLK, F_HID), lambda i: (i, 0)),
            pl.BlockSpec((_BLK, 128), lambda i: (i, 0)),
            pl.BlockSpec((_BLK, 128), lambda i: (i, 0)),
        ],
        out_shape=[
            jax.ShapeDtypeStruct((N_PAD, F_HID), jnp.float32),
            jax.ShapeDtypeStruct((N_PAD, 128), jnp.float32),
            jax.ShapeDtypeStruct((N_PAD, 128), jnp.float32),
        ],
    )(a0, a1, degr, b1, W2)


def _stage_c(a0, a1, degr, b2, z, W3):
    def body(a0_ref, a1_ref, d_ref, b_ref, z_ref, w_ref, o):
        dv = _dinv(d_ref[...])
        m = jnp.concatenate([a0_ref[...] * dv, a1_ref[...] * dv], axis=1)
        h2 = jnp.maximum(m + b_ref[...] + z_ref[...], 0.0)
        g = jnp.dot(h2, w_ref[...], preferred_element_type=jnp.float32)
        gs = g * dv[:, :F_OUT]
        o[...] = jnp.concatenate([gs, jnp.zeros_like(gs)], axis=1)

    return pl.pallas_call(
        body,
        grid=(_GRID,),
        in_specs=[
            pl.BlockSpec((_BLK, 128), lambda i: (i, 0)),
            pl.BlockSpec((_BLK, 128), lambda i: (i, 0)),
            pl.BlockSpec((_BLK, 128), lambda i: (i, 0)),
            pl.BlockSpec((1, F_HID), lambda i: (0, 0)),
            pl.BlockSpec((_BLK, F_HID), lambda i: (i, 0)),
            pl.BlockSpec((F_HID, F_OUT), lambda i: (0, 0)),
        ],
        out_specs=pl.BlockSpec((_BLK, 128), lambda i: (i, 0)),
        out_shape=jax.ShapeDtypeStruct((N_PAD, 128), jnp.float32),
    )(a0, a1, degr, b2, z, W3)


def _stage_d(a0, a1, degr, b3):
    def body(a0_ref, a1_ref, d_ref, b_ref, o):
        dv = _dinv(d_ref[...])
        agg = a0_ref[...] + a1_ref[...]
        o[...] = agg[:, :F_OUT] * dv[:, :F_OUT] + b_ref[...]

    return pl.pallas_call(
        body,
        grid=(_GRID,),
        in_specs=[
            pl.BlockSpec((_BLK, 128), lambda i: (i, 0)),
            pl.BlockSpec((_BLK, 128), lambda i: (i, 0)),
            pl.BlockSpec((_BLK, 128), lambda i: (i, 0)),
            pl.BlockSpec((1, F_OUT), lambda i: (0, 0)),
        ],
        out_specs=pl.BlockSpec((_BLK, F_OUT), lambda i: (i, 0)),
        out_shape=jax.ShapeDtypeStruct((N_PAD, F_OUT), jnp.float32),
    )(a0, a1, degr, b3)


_deg_kernel = _make_deg_kernel()
_agg_hid = _make_agg_kernel(F_HID // 2)
_agg3 = _make_agg3_kernel()


def kernel(x, edge_index, W1, b1, W2, b2, W3, b3):
    ei = edge_index.astype(jnp.int32)
    src = ei[0]
    dst = ei[1]
    pad = E_PAD - N_EDGES
    srcp = jnp.concatenate([src, jnp.zeros((pad,), jnp.int32)])
    dstp = jnp.concatenate([dst, jnp.full((pad,), DUMMY_ROW, jnp.int32)])
    src16 = srcp.reshape(16, N_CHUNKS, CHUNK)
    dst16 = dstp.reshape(16, N_CHUNKS, CHUNK)

    xp = jnp.pad(x, ((0, N_PAD - N_NODES), (0, 0)))

    deg = _deg_kernel(dst16)
    degr = jnp.broadcast_to(deg[:, None], (N_PAD, 128))

    h0, h1 = _stage_a(xp, W1, degr)
    a0, a1 = _agg_hid(h0, h1, src16, dst16)
    z, g0, g1 = _stage_b(a0, a1, degr, b1.reshape(1, F_HID), W2)
    a0, a1 = _agg_hid(g0, g1, src16, dst16)
    g3 = _stage_c(a0, a1, degr, b2.reshape(1, F_HID), z, W3)
    a0, a1 = _agg3(g3, src16, dst16)
    out = _stage_d(a0, a1, degr, b3.reshape(1, F_OUT))
    return out[:N_NODES]
